# Initial kernel scaffold; baseline (speedup 1.0000x reference)
import jax
import jax.numpy as jnp
from jax.experimental import pallas as pl


def kernel(x, offset, mask, W):
    raise NotImplementedError("write your pallas kernel here")


# speedup vs baseline: 124.5280x; 124.5280x over previous
import functools

import numpy as np
import jax
import jax.numpy as jnp
from jax import lax
from jax.experimental import pallas as pl
from jax.experimental.pallas import tpu as pltpu
from jax.experimental.pallas import tpu_sc as plsc

_B = 256
_C = 3
_H = 224
_Wd = 224
_RH = 18
_RW = 18
_K = 9
_OH = 16
_OW = 16
_NW = 32
_BPW = _B // _NW
_PLANE = _RH * _RW
_IMG = _C * _PLANE
_OFFS = 2 * _K * _OH * _OW
_MSKS = _K * _OH * _OW
_OUTS = _OH * _OW
_BS = 8


def _interp_matrix(out_n, in_n):
    s = in_n / out_n
    pos = np.maximum((np.arange(out_n) + 0.5) * s - 0.5, 0.0).astype(np.float32)
    i0 = np.floor(pos).astype(np.int32)
    i1 = np.minimum(i0 + 1, in_n - 1)
    f = (pos - i0).astype(np.float32)
    m = np.zeros((out_n, in_n), np.float32)
    m[np.arange(out_n), i0] += 1.0 - f
    m[np.arange(out_n), i1] += f
    return m


def _resize_body(x_ref, ry_ref, rxt_ref, o_ref):
    ry = ry_ref[...]
    rxt = rxt_ref[...]
    xb = x_ref[...].reshape(_BS * _C, _H, _Wd)
    rows = [jnp.dot(ry, xb[t], preferred_element_type=jnp.float32)
            for t in range(_BS * _C)]
    xr = jnp.concatenate(rows, axis=0)
    xc = jnp.dot(xr, rxt, preferred_element_type=jnp.float32)
    o_ref[...] = xc.reshape(_BS, _C, _RH, _RW)


def _resize(x):
    ry = jnp.asarray(_interp_matrix(_RH, _H))
    rxt = jnp.asarray(_interp_matrix(_RW, _Wd).T)
    return pl.pallas_call(
        _resize_body,
        grid=(_B // _BS,),
        in_specs=[
            pl.BlockSpec((_BS, _C, _H, _Wd), lambda i: (i, 0, 0, 0)),
            pl.BlockSpec((_RH, _H), lambda i: (0, 0)),
            pl.BlockSpec((_Wd, _RW), lambda i: (0, 0)),
        ],
        out_specs=pl.BlockSpec((_BS, _C, _RH, _RW), lambda i: (i, 0, 0, 0)),
        out_shape=jax.ShapeDtypeStruct((_B, _C, _RH, _RW), jnp.float32),
    )(x, ry, rxt)


def _deform_sc(inp_flat, off_flat, mask_flat, w_pad):
    mesh = plsc.VectorSubcoreMesh(core_axis_name="c", subcore_axis_name="s")

    @functools.partial(
        pl.kernel,
        mesh=mesh,
        compiler_params=pltpu.CompilerParams(needs_layout_passes=False),
        out_type=jax.ShapeDtypeStruct((_B * _OUTS,), jnp.float32),
        scratch_types=[
            pltpu.VMEM((_BPW * _IMG,), jnp.float32),
            pltpu.VMEM((_BPW * _OFFS,), jnp.float32),
            pltpu.VMEM((_BPW * _MSKS,), jnp.float32),
            pltpu.VMEM((32,), jnp.float32),
            pltpu.VMEM((_BPW * _OUTS,), jnp.float32),
        ],
    )
    def _deform(inp_hbm, off_hbm, mask_hbm, w_hbm, out_hbm,
                inp_v, off_v, mask_v, w_v, out_v):
        wid = lax.axis_index("s") * 2 + lax.axis_index("c")
        pltpu.sync_copy(inp_hbm.at[pl.ds(wid * (_BPW * _IMG), _BPW * _IMG)], inp_v)
        pltpu.sync_copy(off_hbm.at[pl.ds(wid * (_BPW * _OFFS), _BPW * _OFFS)], off_v)
        pltpu.sync_copy(mask_hbm.at[pl.ds(wid * (_BPW * _MSKS), _BPW * _MSKS)], mask_v)
        pltpu.sync_copy(w_hbm, w_v)

        lanes_f = lax.iota(jnp.int32, 16).astype(jnp.float32)
        zero16 = jnp.zeros((16,), jnp.float32)
        one16 = jnp.ones((16,), jnp.float32)

        def body_b(i, c0):
            def body_v(v, c1):
                def body_k(k, acc):
                    ki = k // 3
                    kj = k % 3
                    obase = i * _OFFS + 2 * k * _OUTS + v * _OW
                    dy = off_v[pl.ds(obase, 16)]
                    dx = off_v[pl.ds(obase + _OUTS, 16)]
                    m = mask_v[pl.ds(i * _MSKS + k * _OUTS + v * _OW, 16)]
                    yy = dy + (v + ki).astype(jnp.float32)
                    xx = dx + kj.astype(jnp.float32) + lanes_f
                    ty = yy.astype(jnp.int32)
                    y0 = ty - jnp.where(ty.astype(jnp.float32) > yy, 1, 0)
                    fy = yy - y0.astype(jnp.float32)
                    tx = xx.astype(jnp.int32)
                    x0 = tx - jnp.where(tx.astype(jnp.float32) > xx, 1, 0)
                    fx = xx - x0.astype(jnp.float32)
                    y1 = y0 + 1
                    x1 = x0 + 1
                    vy0 = jnp.where((y0 >= 0) & (y0 < _RH), one16, zero16)
                    vy1 = jnp.where((y1 >= 0) & (y1 < _RH), one16, zero16)
                    vx0 = jnp.where((x0 >= 0) & (x0 < _RW), one16, zero16)
                    vx1 = jnp.where((x1 >= 0) & (x1 < _RW), one16, zero16)
                    cy0 = jnp.clip(y0, 0, _RH - 1)
                    cy1 = jnp.clip(y1, 0, _RH - 1)
                    cx0 = jnp.clip(x0, 0, _RW - 1)
                    cx1 = jnp.clip(x1, 0, _RW - 1)
                    gy0 = vy0 * (1.0 - fy)
                    gy1 = vy1 * fy
                    gx0 = vx0 * (1.0 - fx)
                    gx1 = vx1 * fx
                    w00 = gy0 * gx0
                    w01 = gy0 * gx1
                    w10 = gy1 * gx0
                    w11 = gy1 * gx1
                    r0 = cy0 * _RW
                    r1 = cy1 * _RW
                    i00 = r0 + cx0
                    i01 = r0 + cx1
                    i10 = r1 + cx0
                    i11 = r1 + cx1
                    ib = i * _IMG
                    tot = zero16
                    for c in range(_C):
                        base = ib + c * _PLANE
                        v00 = plsc.load_gather(inp_v, [i00 + base])
                        v01 = plsc.load_gather(inp_v, [i01 + base])
                        v10 = plsc.load_gather(inp_v, [i10 + base])
                        v11 = plsc.load_gather(inp_v, [i11 + base])
                        s = w00 * v00 + w01 * v01 + w10 * v10 + w11 * v11
                        wk = plsc.load_gather(
                            w_v, [jnp.full((16,), c * _K, jnp.int32) + k])
                        tot = tot + wk * s
                    return acc + m * tot

                acc = lax.fori_loop(0, _K, body_k, zero16)
                out_v[pl.ds(i * _OUTS + v * _OW, 16)] = acc
                return c1

            lax.fori_loop(0, _OH, body_v, 0)
            return c0

        lax.fori_loop(0, _BPW, body_b, 0)
        pltpu.sync_copy(out_v, out_hbm.at[pl.ds(wid * (_BPW * _OUTS), _BPW * _OUTS)])

    return _deform(inp_flat, off_flat, mask_flat, w_pad)


def kernel(x, offset, mask, W):
    inp = _resize(x)
    out_flat = _deform_sc(
        inp.reshape(_B * _IMG),
        offset.reshape(_B * _OFFS),
        mask.reshape(_B * _MSKS),
        jnp.pad(W.reshape(_C * _K), (0, 32 - _C * _K)),
    )
    return out_flat.reshape(_B, 1, _OH, _OW)
